# Initial kernel scaffold; baseline (speedup 1.0000x reference)
import jax
import jax.numpy as jnp
from jax.experimental import pallas as pl


def kernel(species, rad_distances, rad_switch, edge_src, edge_dst, ang_distances, ang_switch, ang_edge_dst, angles, central_atom, angle_src, angle_dst):
    raise NotImplementedError("write your pallas kernel here")


# speedup vs baseline: 34.4010x; 34.4010x over previous
import functools
import numpy as np
import jax
import jax.numpy as jnp
from jax import lax
from jax.experimental import pallas as pl
from jax.experimental.pallas import tpu as pltpu
from jax.experimental.pallas import tpu_sc as plsc

NN = 50000
ER = 1600000
EA = 800000
NA = 1600000
RETA = 16.0
AETA = 8.0
ZETA = 32.0
RSTART, RCUT = 0.8, 5.2
ASTART, ACUT = 0.8, 3.5

_DSCALE = 0.5 * np.sqrt(AETA)
_SWSCALE = np.sqrt(2.0 * 0.5 ** ZETA)
_DQ = 65536.0 / 5.0
_SQ = 16384.0

R_SPLIT = 100096
R_SPAN = R_SPLIT // 16
AP_ATOMS = 10112
AP_ROWS = AP_ATOMS * 10
A_SPAN = AP_ROWS // 16
ACC_ROWS = 102400
A_BASE = 2 * R_SPLIT
OUT_ROWS = A_BASE + 5 * AP_ROWS

CHUNK = 2000
SCHUNK = 800
GROUPS = CHUNK // 16


def _sidx(sp):
    return (jnp.where(sp >= 6, 1, 0) + jnp.where(sp >= 7, 1, 0)
            + jnp.where(sp >= 8, 1, 0))


def _pre_body(ad_hbm, asw_hbm, adst_hbm, spec_hbm, cen_hbm, asrc_hbm,
              adst2_hbm, esrc_hbm, edst_hbm,
              d12_hbm, sp_hbm, aidx_hbm, ridx_hbm,
              tab,
              src_b, dst_b, ridx_b, spg_b,
              ad_b, asw_b, adst_b, tout_b,
              asrc_b, adst2_b, cen_b, u1_b, u2_b,
              d12_b, spp_b, aidx_b,
              g0, g1, g2):
    c = lax.axis_index("c")
    s = lax.axis_index("s")
    w = c * 16 + s

    def rad_chunk(i, _):
        base = w * (ER // 32) + i * CHUNK
        pltpu.sync_copy(esrc_hbm.at[pl.ds(base, CHUNK)], src_b)
        pltpu.sync_copy(edst_hbm.at[pl.ds(base, CHUNK)], dst_b)
        pltpu.async_copy(spec_hbm.at[dst_b], spg_b, g0).wait()

        def grp(g, _):
            si = _sidx(spg_b[pl.ds(g * 16, 16)])
            ridx_b[pl.ds(g * 16, 16)] = src_b[pl.ds(g * 16, 16)] * 4 + si
            return _
        lax.fori_loop(0, GROUPS, grp, None)
        pltpu.sync_copy(ridx_b, ridx_hbm.at[pl.ds(base, CHUNK)])
        return _
    lax.fori_loop(0, (ER // 32) // CHUNK, rad_chunk, None)

    def tab_chunk(i, _):
        base = s * (EA // 16) + i * CHUNK
        pltpu.sync_copy(ad_hbm.at[pl.ds(base, CHUNK)], ad_b)
        pltpu.sync_copy(asw_hbm.at[pl.ds(base, CHUNK)], asw_b)
        pltpu.sync_copy(adst_hbm.at[pl.ds(base, CHUNK)], adst_b)
        pltpu.async_copy(spec_hbm.at[adst_b], spg_b, g0).wait()

        def grp(g, _):
            sl = pl.ds(g * 16, 16)
            si = _sidx(spg_b[sl])
            dist = ad_b[sl] * np.float32(_DSCALE)
            q16 = jnp.minimum((dist * np.float32(_DQ) + 0.5)
                              .astype(jnp.int32), 65535)
            q14 = jnp.minimum((asw_b[sl] * np.float32(_SQ) + 0.5)
                              .astype(jnp.int32), 16383)
            tout_b[sl] = (q16 << 16) | (q14 << 2) | si
            return _
        lax.fori_loop(0, GROUPS, grp, None)
        pltpu.sync_copy(tout_b, tab.at[pl.ds(base, CHUNK)])
        return _
    lax.fori_loop(0, (EA // 16) // CHUNK, tab_chunk, None)

    plsc.subcore_barrier()

    def tri_chunk(i, _):
        base = c * (NA // 2) + s * (NA // 32) + i * CHUNK
        pltpu.sync_copy(asrc_hbm.at[pl.ds(base, CHUNK)], asrc_b)
        pltpu.sync_copy(adst2_hbm.at[pl.ds(base, CHUNK)], adst2_b)
        pltpu.sync_copy(cen_hbm.at[pl.ds(base, CHUNK)], cen_b)
        cp1 = pltpu.async_copy(tab.at[asrc_b], u1_b, g1)
        cp2 = pltpu.async_copy(tab.at[adst2_b], u2_b, g2)
        cp1.wait()
        cp2.wait()

        def grp(g, _):
            sl = pl.ds(g * 16, 16)
            u1 = u1_b[sl]
            u2 = u2_b[sl]
            d1 = lax.shift_right_logical(u1, 16).astype(jnp.float32)
            d2 = lax.shift_right_logical(u2, 16).astype(jnp.float32)
            d12_b[sl] = (d1 + d2) * np.float32(5.0 / 65536.0)
            s1 = (lax.shift_right_logical(u1, 2) & 16383).astype(jnp.float32)
            s2 = (lax.shift_right_logical(u2, 2) & 16383).astype(jnp.float32)
            spp_b[sl] = s1 * s2 * np.float32((_SWSCALE / _SQ) ** 2)
            i1 = u1 & 3
            i2 = u2 & 3
            a = jnp.minimum(i1, i2)
            b = jnp.maximum(i1, i2)
            pair = ((a * (7 - a)) >> 1) + b
            aidx_b[sl] = cen_b[sl] * 10 + pair
            return _
        lax.fori_loop(0, GROUPS, grp, None)
        pltpu.sync_copy(d12_b, d12_hbm.at[pl.ds(base, CHUNK)])
        pltpu.sync_copy(spp_b, sp_hbm.at[pl.ds(base, CHUNK)])
        pltpu.sync_copy(aidx_b, aidx_hbm.at[pl.ds(base, CHUNK)])
        return _
    lax.fori_loop(0, (NA // 32) // CHUNK, tri_chunk, None)


@jax.jit
def _pre(ad, asw, adst, spec, cen, asrc, adst2, esrc, edst):
    f32, i32 = jnp.float32, jnp.int32
    mesh = plsc.VectorSubcoreMesh(core_axis_name="c", subcore_axis_name="s")
    return pl.kernel(
        _pre_body,
        out_type=[
            jax.ShapeDtypeStruct((NA,), f32),
            jax.ShapeDtypeStruct((NA,), f32),
            jax.ShapeDtypeStruct((NA,), i32),
            jax.ShapeDtypeStruct((ER,), i32),
        ],
        mesh=mesh,
        scratch_types=[
            pltpu.VMEM_SHARED((EA,), i32),
            pltpu.VMEM((CHUNK,), i32), pltpu.VMEM((CHUNK,), i32),
            pltpu.VMEM((CHUNK,), i32), pltpu.VMEM((CHUNK,), i32),
            pltpu.VMEM((CHUNK,), f32), pltpu.VMEM((CHUNK,), f32),
            pltpu.VMEM((CHUNK,), i32), pltpu.VMEM((CHUNK,), i32),
            pltpu.VMEM((CHUNK,), i32), pltpu.VMEM((CHUNK,), i32),
            pltpu.VMEM((CHUNK,), i32),
            pltpu.VMEM((CHUNK,), i32), pltpu.VMEM((CHUNK,), i32),
            pltpu.VMEM((CHUNK,), f32), pltpu.VMEM((CHUNK,), f32),
            pltpu.VMEM((CHUNK,), i32),
            pltpu.SemaphoreType.DMA, pltpu.SemaphoreType.DMA,
            pltpu.SemaphoreType.DMA,
        ],
        compiler_params=pltpu.CompilerParams(needs_layout_passes=False, use_tc_tiling_on_sc=False),
    )(ad, asw, adst, spec, cen, asrc, adst2, esrc, edst)


def _lane_mod(n, m):
    k = lax.broadcasted_iota(jnp.int32, (1, 128), 1)
    return (k % n).astype(jnp.float32) * np.float32(m)


def _rad_terms_body(d_ref, sw_ref, m_ref, o_ref):
    m = m_ref[...]
    dexp = jnp.dot(d_ref[...], m, preferred_element_type=jnp.float32, precision=lax.Precision.HIGHEST)
    swexp = jnp.dot(sw_ref[...], m, preferred_element_type=jnp.float32, precision=lax.Precision.HIGHEST)
    shift = -(np.float32(RSTART) + _lane_mod(16, (RCUT - RSTART) / 16.0))
    x = dexp + shift
    o_ref[...] = jnp.exp(-(np.float32(RETA) * x * x)) * (0.25 * swexp)


@jax.jit
def _rad_terms(d8, sw8, m8):
    br, grid = 2000, (ER // 8) // 2000
    return pl.pallas_call(
        _rad_terms_body,
        grid=(grid,),
        in_specs=[
            pl.BlockSpec((br, 8), lambda i: (i, 0)),
            pl.BlockSpec((br, 8), lambda i: (i, 0)),
            pl.BlockSpec((8, 128), lambda i: (0, 0)),
        ],
        out_specs=pl.BlockSpec((br, 128), lambda i: (i, 0)),
        out_shape=jax.ShapeDtypeStruct((ER // 8, 128), jnp.float32),
    )(d8, sw8, m8)


def _ang_factors_body(th_ref, sp_ref, dd_ref, m_ref, f1_ref, f2_ref):
    m = m_ref[...]
    th = jnp.dot(th_ref[...], m, preferred_element_type=jnp.float32, precision=lax.Precision.HIGHEST)
    spx = jnp.dot(sp_ref[...], m, preferred_element_type=jnp.float32, precision=lax.Precision.HIGHEST)
    ddx = jnp.dot(dd_ref[...], m, preferred_element_type=jnp.float32, precision=lax.Precision.HIGHEST)
    tz = -(np.float32(np.pi / 8.0) + _lane_mod(4, np.pi / 4.0))
    ta = -np.float32(np.sqrt(AETA)) * (np.float32(ASTART)
                                       + _lane_mod(4, (ACUT - ASTART) / 4.0))
    f = 1.0 + jnp.cos(th + tz)
    f = f * f
    f = f * f
    f = f * f
    f = f * f
    f = f * f
    f1_ref[...] = f * spx
    y = ddx + ta
    f2_ref[...] = jnp.exp(-(y * y))


@jax.jit
def _ang_factors(th32, sp32, dd32, m32):
    br, grid = 1000, (NA // 32) // 1000
    spec_in = pl.BlockSpec((br, 32), lambda i: (i, 0))
    spec_out = pl.BlockSpec((br, 128), lambda i: (i, 0))
    return pl.pallas_call(
        _ang_factors_body,
        grid=(grid,),
        in_specs=[spec_in, spec_in, spec_in,
                  pl.BlockSpec((32, 128), lambda i: (0, 0))],
        out_specs=[spec_out, spec_out],
        out_shape=[jax.ShapeDtypeStruct((NA // 32, 128), jnp.float32),
                   jax.ShapeDtypeStruct((NA // 32, 128), jnp.float32)],
    )(th32, sp32, dd32, m32)


def _ang_expand_body(f1_ref, f2_ref, mb_ref, ma_ref, o_ref):
    fz = jnp.dot(f1_ref[...], mb_ref[...], preferred_element_type=jnp.float32, precision=lax.Precision.HIGHEST)
    fa = jnp.dot(f2_ref[...], ma_ref[...], preferred_element_type=jnp.float32, precision=lax.Precision.HIGHEST)
    o_ref[...] = fa * fz


@jax.jit
def _ang_expand(f1v, f2v, mb, ma):
    br, grid = 2000, (NA // 8) // 2000
    spec_in = pl.BlockSpec((br, 32), lambda i: (i, 0))
    return pl.pallas_call(
        _ang_expand_body,
        grid=(grid,),
        in_specs=[spec_in, spec_in,
                  pl.BlockSpec((32, 128), lambda i: (0, 0)),
                  pl.BlockSpec((32, 128), lambda i: (0, 0))],
        out_specs=pl.BlockSpec((br, 128), lambda i: (i, 0)),
        out_shape=jax.ShapeDtypeStruct((NA // 8, 128), jnp.float32),
    )(f1v, f2v, mb, ma)


def _scatter_body(ridx_hbm, rt_hbm, aidx_hbm, at_hbm, zeros_hbm, out_hbm,
                  acc, zb, ib, tb, rb):
    c = lax.axis_index("c")
    s = lax.axis_index("s")

    pltpu.sync_copy(zeros_hbm, zb)

    def zero_acc():
        def z(i, _):
            pltpu.sync_copy(
                zb, acc.at[pl.ds(s * (ACC_ROWS // 16) + i * 200, 200)])
            return _
        lax.fori_loop(0, ACC_ROWS // 16 // 200, z, None)

    def stream(idx_hbm, t_hbm, rows, base):
        def chunk(i, _):
            eb = s * (ER // 16) + i * SCHUNK
            pltpu.sync_copy(idx_hbm.at[pl.ds(eb, SCHUNK)], ib)
            pltpu.sync_copy(t_hbm.at[pl.ds(eb, SCHUNK)], tb)

            def grp(g, _):
                sl = pl.ds(g * 16, 16)
                v = ib[sl]
                local = v - base
                ok = (local >= 0) & (local < rows)
                dump = rows + (v & 1023)
                ib[sl] = jnp.where(ok, local, dump)
                return _
            lax.fori_loop(0, SCHUNK // 16, grp, None)
            pltpu.sync_copy(tb, acc.at[ib], add=True)
            return _
        lax.fori_loop(0, (ER // 16) // SCHUNK, chunk, None)

    def readout(span, bounce, out_base):
        def ro(i, _):
            r0 = s * span + i * bounce
            pltpu.sync_copy(acc.at[pl.ds(r0, bounce)],
                            rb.at[pl.ds(0, bounce)])
            pltpu.sync_copy(rb.at[pl.ds(0, bounce)],
                            out_hbm.at[pl.ds(out_base + r0, bounce)])
            return _
        lax.fori_loop(0, span // bounce, ro, None)

    zero_acc()
    plsc.subcore_barrier()
    stream(ridx_hbm, rt_hbm, R_SPLIT - c * 192, c * R_SPLIT)
    plsc.subcore_barrier()
    readout(R_SPAN, 368, c * R_SPLIT)
    plsc.subcore_barrier()

    def ang_pass(p, _):
        q = 2 * p + c
        zero_acc()
        plsc.subcore_barrier()

        @pl.when(q <= 4)
        def _():
            stream(aidx_hbm, at_hbm, AP_ROWS, q * AP_ROWS)
        plsc.subcore_barrier()

        @pl.when(q <= 4)
        def _():
            readout(A_SPAN, 80, A_BASE + q * AP_ROWS)
        plsc.subcore_barrier()
        return _
    lax.fori_loop(0, 3, ang_pass, None)


@jax.jit
def _scatter(ridx, rt, aidx, at, zeros_in):
    f32 = jnp.float32
    mesh = plsc.VectorSubcoreMesh(core_axis_name="c", subcore_axis_name="s")
    return pl.kernel(
        _scatter_body,
        out_type=jax.ShapeDtypeStruct((OUT_ROWS, 16), f32),
        mesh=mesh,
        scratch_types=[
            pltpu.VMEM_SHARED((ACC_ROWS, 16), f32),
            pltpu.VMEM((200, 16), f32),
            pltpu.VMEM((SCHUNK,), jnp.int32),
            pltpu.VMEM((SCHUNK, 16), f32),
            pltpu.VMEM((368, 16), f32),
        ],
        compiler_params=pltpu.CompilerParams(needs_layout_passes=False, use_tc_tiling_on_sc=False),
    )(ridx, rt, aidx, at, zeros_in)


def _expand_mats():
    m8 = np.zeros((8, 128), np.float32)
    for e in range(8):
        m8[e, e * 16:(e + 1) * 16] = 1.0
    m32 = np.zeros((32, 128), np.float32)
    for m in range(32):
        m32[m, m * 4:(m + 1) * 4] = 1.0
    mb = np.zeros((32, 128), np.float32)
    ma = np.zeros((32, 128), np.float32)
    for e in range(8):
        for a in range(4):
            for z in range(4):
                mb[e * 4 + z, e * 16 + a * 4 + z] = 1.0
                ma[e * 4 + a, e * 16 + a * 4 + z] = 1.0
    return (jnp.asarray(m8), jnp.asarray(m32), jnp.asarray(mb),
            jnp.asarray(ma))


def kernel(species, rad_distances, rad_switch, edge_src, edge_dst,
           ang_distances, ang_switch, ang_edge_dst, angles, central_atom,
           angle_src, angle_dst):
    m8, m32, mb, ma = _expand_mats()
    d12, sprod, aidx, ridx = _pre(
        ang_distances, ang_switch, ang_edge_dst, species, central_atom,
        angle_src, angle_dst, edge_src, edge_dst)
    rt = _rad_terms(rad_distances.reshape(ER // 8, 8),
                    rad_switch.reshape(ER // 8, 8), m8)
    f1, f2 = _ang_factors(angles.reshape(NA // 32, 32),
                          sprod.reshape(NA // 32, 32),
                          d12.reshape(NA // 32, 32), m32)
    at = _ang_expand(f1.reshape(NA // 8, 32), f2.reshape(NA // 8, 32), mb, ma)
    zeros_in = jnp.zeros((200, 16), jnp.float32)
    out = _scatter(ridx, rt.reshape(ER, 16), aidx, at.reshape(NA, 16),
                   zeros_in)
    radial = out[:4 * NN].reshape(NN, 64)
    angular = out[A_BASE:A_BASE + 10 * NN].reshape(NN, 160)
    return jnp.concatenate([radial, angular], axis=-1)
